# Initial kernel scaffold; baseline (speedup 1.0000x reference)
import jax
import jax.numpy as jnp
from jax.experimental import pallas as pl


def kernel(x_all, mask, template_edge_index, params):
    raise NotImplementedError("write your pallas kernel here")


# speedup vs baseline: 11.8540x; 11.8540x over previous
import functools
import jax
import jax.numpy as jnp
import numpy as np
from jax import lax
from jax.experimental import pallas as pl
from jax.experimental.pallas import tpu as pltpu
from jax.experimental.pallas import tpu_sc as plsc

N = 5000
B = 8
HID = 128
CARDS = [12, 24, 48, 96, 365, 1000, 5000, 20000]
EMB_DIMS = [min(50, (c + 1) // 2) for c in CARDS]
NTAB = len(CARDS)
DPAD = 128
CHUNK = 128
NCH = 43
EPAD = 16 * NCH * CHUNK
NACC = 5120
RPT = NACC // 16
NGCH = 79
IPAD = 32 * NGCH * 128
RB = 1000
NROW = B * N


def _mesh():
    return plsc.VectorSubcoreMesh(core_axis_name="c", subcore_axis_name="s")


def _deg_call(dst_p, ones_hbm, zeros_hbm):
    @functools.partial(
        pl.kernel,
        out_type=jax.ShapeDtypeStruct((NACC, 128), jnp.float32),
        mesh=_mesh(),
        scratch_types=[
            pltpu.VMEM((NCH, CHUNK), jnp.int32),
            pltpu.VMEM((CHUNK, 128), jnp.float32),
            pltpu.VMEM((64, 128), jnp.float32),
            pltpu.VMEM_SHARED((NACC, 128), jnp.float32),
        ],
    )
    def deg_kernel(dst_hbm, ones_h, zeros_h, out_hbm, dstbuf, ones_v, zbuf, acc):
        c = lax.axis_index("c")
        s = lax.axis_index("s")

        @pl.when(c == 0)
        def _():
            pltpu.sync_copy(ones_h, ones_v)
            pltpu.sync_copy(zeros_h, zbuf)
            pltpu.sync_copy(dst_hbm.at[s], dstbuf)
            for k in range(5):
                pltpu.sync_copy(zbuf, acc.at[pl.ds(s * RPT + k * 64, 64)])
            plsc.subcore_barrier()

            def body(j, carry):
                pltpu.sync_copy(ones_v, acc.at[dstbuf.at[j]], add=True)
                return carry

            lax.fori_loop(0, NCH, body, 0)
            plsc.subcore_barrier()
            pltpu.sync_copy(acc.at[pl.ds(s * RPT, RPT)],
                            out_hbm.at[pl.ds(s * RPT, RPT)])

    return deg_kernel(dst_p, ones_hbm, zeros_hbm)


def _emb_call(big, idxp):
    @functools.partial(
        pl.kernel,
        out_type=jax.ShapeDtypeStruct((IPAD, DPAD), jnp.float32),
        mesh=_mesh(),
        scratch_types=[
            pltpu.VMEM((NGCH, CHUNK), jnp.int32),
            pltpu.VMEM((CHUNK, DPAD), jnp.float32),
            pltpu.SemaphoreType.DMA,
        ],
    )
    def emb_kernel(tab_hbm, idx_hbm, out_hbm, idxbuf, rows_v, sem):
        c = lax.axis_index("c")
        s = lax.axis_index("s")
        wid = s * 2 + c
        pltpu.sync_copy(idx_hbm.at[wid], idxbuf)

        def body(j, carry):
            pltpu.async_copy(tab_hbm.at[idxbuf.at[j]], rows_v, sem).wait()
            pltpu.sync_copy(rows_v,
                            out_hbm.at[pl.ds((wid * NGCH + j) * CHUNK, CHUNK)])
            return carry

        lax.fori_loop(0, NGCH, body, 0)

    return emb_kernel(big, idxp)


def _agg_call(y2, srcabs_p, dst_p, zeros_hbm):
    @functools.partial(
        pl.kernel,
        out_type=jax.ShapeDtypeStruct((B * 2 * NACC, HID), jnp.float32),
        mesh=_mesh(),
        scratch_types=[
            pltpu.VMEM((NCH, CHUNK), jnp.int32),
            pltpu.VMEM((NCH, CHUNK), jnp.int32),
            pltpu.VMEM((CHUNK, HID), jnp.float32),
            pltpu.VMEM((64, HID), jnp.float32),
            pltpu.VMEM_SHARED((NACC, HID), jnp.float32),
            pltpu.SemaphoreType.DMA,
        ],
    )
    def agg_kernel(y_hbm, srcabs_hbm, dst_hbm, zeros_h, out_hbm,
                   idxbuf, dstbuf, rows_v, zbuf, acc, sem):
        c = lax.axis_index("c")
        s = lax.axis_index("s")
        pltpu.sync_copy(zeros_h, zbuf)
        pltpu.sync_copy(dst_hbm.at[s], dstbuf)
        for bi in range(4):
            for h in range(2):
                bh = (c * 4 + bi) * 2 + h
                pltpu.sync_copy(srcabs_hbm.at[bh * 16 + s], idxbuf)
                for k in range(5):
                    pltpu.sync_copy(zbuf, acc.at[pl.ds(s * RPT + k * 64, 64)])
                plsc.subcore_barrier()

                def body(j, carry):
                    pltpu.async_copy(y_hbm.at[idxbuf.at[j]], rows_v, sem).wait()
                    pltpu.sync_copy(rows_v, acc.at[dstbuf.at[j]], add=True)
                    return carry

                lax.fori_loop(0, NCH, body, 0)
                plsc.subcore_barrier()
                pltpu.sync_copy(acc.at[pl.ds(s * RPT, RPT)],
                                out_hbm.at[pl.ds(bh * NACC + s * RPT, RPT)])
                plsc.subcore_barrier()

    return agg_kernel(y2, srcabs_p, dst_p, zeros_hbm)


def _mm_kernel(x_ref, w_ref, s_ref, o_ref):
    o_ref[...] = jnp.dot(x_ref[...], w_ref[...],
                         preferred_element_type=jnp.float32) * s_ref[...]


def _tc_mm(x, w, sv):
    k = x.shape[1]
    return pl.pallas_call(
        _mm_kernel,
        grid=(NROW // RB,),
        in_specs=[
            pl.BlockSpec((RB, k), lambda i: (i, 0)),
            pl.BlockSpec((k, 2 * HID), lambda i: (0, 0)),
            pl.BlockSpec((RB, 1), lambda i: (i, 0)),
        ],
        out_specs=pl.BlockSpec((RB, 2 * HID), lambda i: (i, 0)),
        out_shape=jax.ShapeDtypeStruct((NROW, 2 * HID), jnp.float32),
    )(x, w, sv)


def _gate_kernel(ay_ref, s_ref, bc_ref, lz_ref, lh_ref, b2_ref, o_ref, os_ref):
    czh = ay_ref[...] * s_ref[...] + bc_ref[...]
    z = jax.nn.sigmoid(
        jnp.dot(czh[:, :HID], lz_ref[...], preferred_element_type=jnp.float32)
        + b2_ref[:, :HID])
    ht = jnp.tanh(
        jnp.dot(czh[:, HID:], lh_ref[...], preferred_element_type=jnp.float32)
        + b2_ref[:, HID:])
    out = (1.0 - z) * ht
    o_ref[...] = out
    bsum = jnp.sum(out, axis=0, keepdims=True) * 0.125
    os_ref[...] = jnp.broadcast_to(bsum, (8, HID))


def _tc_gate(ay, sv, bcat, lz, lh, b2):
    nb = NROW // RB
    return pl.pallas_call(
        _gate_kernel,
        grid=(nb,),
        in_specs=[
            pl.BlockSpec((RB, 2 * HID), lambda i: (i, 0)),
            pl.BlockSpec((RB, 1), lambda i: (i, 0)),
            pl.BlockSpec((1, 2 * HID), lambda i: (0, 0)),
            pl.BlockSpec((HID, HID), lambda i: (0, 0)),
            pl.BlockSpec((HID, HID), lambda i: (0, 0)),
            pl.BlockSpec((1, 2 * HID), lambda i: (0, 0)),
        ],
        out_specs=[
            pl.BlockSpec((RB, HID), lambda i: (i, 0)),
            pl.BlockSpec((8, HID), lambda i: (i, 0)),
        ],
        out_shape=[
            jax.ShapeDtypeStruct((NROW, HID), jnp.float32),
            jax.ShapeDtypeStruct((nb * 8, HID), jnp.float32),
        ],
    )(ay, sv, bcat, lz, lh, b2)


def _cls_kernel(c_ref, w1_ref, b1_ref, w2_ref, b2_ref, o_ref):
    h = jnp.maximum(
        jnp.dot(c_ref[...], w1_ref[...], preferred_element_type=jnp.float32)
        + b1_ref[...], 0.0)
    o_ref[...] = jnp.dot(h, w2_ref[...],
                         preferred_element_type=jnp.float32) + b2_ref[...]


def _tc_cls(comb, w1, b1, w2p, b2p):
    return pl.pallas_call(
        _cls_kernel,
        in_specs=[pl.BlockSpec(x.shape, lambda: (0,) * x.ndim)
                  for x in (comb, w1, b1, w2p, b2p)],
        out_specs=pl.BlockSpec((B, HID), lambda: (0, 0)),
        out_shape=jax.ShapeDtypeStruct((B, HID), jnp.float32),
    )(comb, w1, b1, w2p, b2p)


def kernel(x_all, mask, template_edge_index, params):
    ei = template_edge_index.astype(jnp.int32)
    loopn = jnp.arange(N, dtype=jnp.int32)
    src = jnp.concatenate([ei[0], loopn])
    dst = jnp.concatenate([ei[1], loopn])
    pad = EPAD - src.shape[0]
    src_p = jnp.concatenate([src, jnp.zeros((pad,), jnp.int32)])
    dst_p = jnp.concatenate([dst, jnp.full((pad,), N, jnp.int32)])
    src_p = src_p.reshape(16, NCH, CHUNK)
    dst_p = dst_p.reshape(16, NCH, CHUNK)
    bh = (jnp.arange(B, dtype=jnp.int32)[:, None] * (2 * N)
          + jnp.arange(2, dtype=jnp.int32)[None, :])
    srcabs = 2 * src_p[None, None] + bh[:, :, None, None, None]
    srcabs = srcabs.reshape(B * 2 * 16, NCH, CHUNK)

    ones16 = jnp.ones((CHUNK, 128), jnp.float32)
    zeros16 = jnp.zeros((64, 128), jnp.float32)
    zeros128 = jnp.zeros((64, HID), jnp.float32)

    degacc = _deg_call(dst_p, ones16, zeros16)
    deg = degacc[:N, 0]
    dinv = deg ** -0.5

    tabs = [params["emb_%d" % i] for i in range(NTAB)]
    big = jnp.concatenate(
        [jnp.pad(t, ((0, 0), (0, DPAD - t.shape[1]))) for t in tabs], axis=0)
    offs = np.concatenate([[0], np.cumsum(CARDS)[:-1]]).astype(np.int32)
    x0 = x_all[0].astype(jnp.int32)
    flat_idx = (x0 + jnp.asarray(offs)[None, :]).reshape(-1)
    flat_idx = jnp.concatenate(
        [flat_idx, jnp.zeros((IPAD - flat_idx.shape[0],), jnp.int32)])
    feats = _emb_call(big, flat_idx.reshape(32, NGCH, CHUNK))
    feats = feats[:NROW * NTAB].reshape(NROW, NTAB * DPAD)

    m0 = mask[0].astype(jnp.float32)
    sv_mask = (m0[:, None] * dinv[None, :]).reshape(NROW, 1)
    sv_plain = jnp.broadcast_to(dinv[None, :], (B, N)).reshape(NROW, 1)

    X = feats
    hs = []
    for l in range(2):
        p = params["layer_%d" % l]
        wc = jnp.concatenate([p["conv_z_W"], p["conv_h_W"]], axis=1)
        if l == 0:
            rows, o = [], 0
            for d in EMB_DIMS:
                rows.append(wc[o:o + d])
                rows.append(jnp.zeros((DPAD - d, 2 * HID), jnp.float32))
                o += d
            wc = jnp.concatenate(rows, axis=0)
            sv = sv_mask
        else:
            sv = sv_plain
        Y = _tc_mm(X, wc, sv)
        AY = _agg_call(Y.reshape(NROW * 2, HID), srcabs, dst_p, zeros128)
        AY = AY.reshape(B, 2, NACC, HID)[:, :, :N, :]
        AY = AY.transpose(0, 2, 1, 3).reshape(NROW, 2 * HID)
        bcat = jnp.concatenate([p["conv_z_b"], p["conv_h_b"]])[None, :]
        b2 = jnp.concatenate([p["lin_z_b"], p["lin_h_b"]])[None, :]
        out, osum = _tc_gate(AY, sv_plain, bcat,
                             p["lin_z_W"][:HID], p["lin_h_W"][:HID], b2)
        hs.append(osum.reshape(B, (N // RB) * 8, HID).sum(axis=1) / N)
        X = out

    comb = jnp.concatenate(hs, axis=1)
    w2p = jnp.pad(params["cls_W2"], ((0, 0), (0, HID - params["cls_W2"].shape[1])))
    b2p = jnp.pad(params["cls_b2"], (0, HID - params["cls_b2"].shape[0]))[None, :]
    res = _tc_cls(comb, params["cls_W1"], params["cls_b1"][None, :], w2p, b2p)
    return res[:, :params["cls_W2"].shape[1]]


# speedup vs baseline: 12.4357x; 1.0491x over previous
import functools
import jax
import jax.numpy as jnp
import numpy as np
from jax import lax
from jax.experimental import pallas as pl
from jax.experimental.pallas import tpu as pltpu
from jax.experimental.pallas import tpu_sc as plsc

N = 5000
B = 8
HID = 128
CARDS = [12, 24, 48, 96, 365, 1000, 5000, 20000]
EMB_DIMS = [min(50, (c + 1) // 2) for c in CARDS]
NTAB = len(CARDS)
DPAD = 128
CHUNK = 128
NCH = 43
EPAD = 16 * NCH * CHUNK
NACC = 5120
RPT = NACC // 16
NGCH = 79
IPAD = 32 * NGCH * 128
RB = 1000
NROW = B * N


def _mesh():
    return plsc.VectorSubcoreMesh(core_axis_name="c", subcore_axis_name="s")


def _deg_call(dst_p, ones_hbm, zeros_hbm):
    @functools.partial(
        pl.kernel,
        out_type=jax.ShapeDtypeStruct((NACC, 128), jnp.float32),
        mesh=_mesh(),
        scratch_types=[
            pltpu.VMEM((NCH, CHUNK), jnp.int32),
            pltpu.VMEM((CHUNK, 128), jnp.float32),
            pltpu.VMEM((64, 128), jnp.float32),
            pltpu.VMEM_SHARED((NACC, 128), jnp.float32),
        ],
    )
    def deg_kernel(dst_hbm, ones_h, zeros_h, out_hbm, dstbuf, ones_v, zbuf, acc):
        c = lax.axis_index("c")
        s = lax.axis_index("s")

        @pl.when(c == 0)
        def _():
            pltpu.sync_copy(ones_h, ones_v)
            pltpu.sync_copy(zeros_h, zbuf)
            pltpu.sync_copy(dst_hbm.at[s], dstbuf)
            for k in range(5):
                pltpu.sync_copy(zbuf, acc.at[pl.ds(s * RPT + k * 64, 64)])
            plsc.subcore_barrier()

            def body(j, carry):
                pltpu.sync_copy(ones_v, acc.at[dstbuf.at[j]], add=True)
                return carry

            lax.fori_loop(0, NCH, body, 0)
            plsc.subcore_barrier()
            pltpu.sync_copy(acc.at[pl.ds(s * RPT, RPT)],
                            out_hbm.at[pl.ds(s * RPT, RPT)])

    return deg_kernel(dst_p, ones_hbm, zeros_hbm)


def _emb_call(big, idxp):
    @functools.partial(
        pl.kernel,
        out_type=jax.ShapeDtypeStruct((IPAD, DPAD), jnp.float32),
        mesh=_mesh(),
        scratch_types=[
            pltpu.VMEM((NGCH, CHUNK), jnp.int32),
            pltpu.VMEM((CHUNK, DPAD), jnp.float32),
            pltpu.VMEM((CHUNK, DPAD), jnp.float32),
            pltpu.VMEM((CHUNK, DPAD), jnp.float32),
            pltpu.VMEM((CHUNK, DPAD), jnp.float32),
            pltpu.SemaphoreType.DMA,
            pltpu.SemaphoreType.DMA,
        ],
    )
    def emb_kernel(tab_hbm, idx_hbm, out_hbm, idxbuf,
                   r0, r1, r2, r3, gsem, ssem):
        c = lax.axis_index("c")
        s = lax.axis_index("s")
        wid = s * 2 + c
        rows = [r0, r1, r2, r3]
        pltpu.sync_copy(idx_hbm.at[wid], idxbuf)

        def group(j0, nb):
            gd = [pltpu.async_copy(tab_hbm.at[idxbuf.at[j0 + u]], rows[u], gsem)
                  for u in range(nb)]
            for d in gd:
                d.wait()
            sd = [pltpu.async_copy(
                      rows[u],
                      out_hbm.at[pl.ds((wid * NGCH + j0 + u) * CHUNK, CHUNK)],
                      ssem)
                  for u in range(nb)]
            for d in sd:
                d.wait()

        def body(g, carry):
            group(g * 4, 4)
            return carry

        lax.fori_loop(0, NGCH // 4, body, 0)
        group((NGCH // 4) * 4, NGCH % 4)

    return emb_kernel(big, idxp)


def _agg_call(y2, srcabs_p, dst_p, zeros_hbm):
    @functools.partial(
        pl.kernel,
        out_type=jax.ShapeDtypeStruct((B * 2 * NACC, HID), jnp.float32),
        mesh=_mesh(),
        scratch_types=[
            pltpu.VMEM((NCH, CHUNK), jnp.int32),
            pltpu.VMEM((NCH, CHUNK), jnp.int32),
            pltpu.VMEM((CHUNK, HID), jnp.float32),
            pltpu.VMEM((CHUNK, HID), jnp.float32),
            pltpu.VMEM((CHUNK, HID), jnp.float32),
            pltpu.VMEM((CHUNK, HID), jnp.float32),
            pltpu.VMEM((64, HID), jnp.float32),
            pltpu.VMEM_SHARED((NACC, HID), jnp.float32),
            pltpu.SemaphoreType.DMA,
            pltpu.SemaphoreType.DMA,
        ],
    )
    def agg_kernel(y_hbm, srcabs_hbm, dst_hbm, zeros_h, out_hbm,
                   idxbuf, dstbuf, r0, r1, r2, r3, zbuf, acc, gsem, ssem):
        c = lax.axis_index("c")
        s = lax.axis_index("s")
        rows = [r0, r1, r2, r3]
        pltpu.sync_copy(zeros_h, zbuf)
        pltpu.sync_copy(dst_hbm.at[s], dstbuf)

        def group(j0, nb):
            gd = [pltpu.async_copy(y_hbm.at[idxbuf.at[j0 + u]], rows[u], gsem)
                  for u in range(nb)]
            for d in gd:
                d.wait()
            sd = [pltpu.async_copy(rows[u], acc.at[dstbuf.at[j0 + u]],
                                   ssem, add=True)
                  for u in range(nb)]
            for d in sd:
                d.wait()

        for bi in range(4):
            for h in range(2):
                bh = (c * 4 + bi) * 2 + h
                pltpu.sync_copy(srcabs_hbm.at[bh * 16 + s], idxbuf)
                for k in range(5):
                    pltpu.sync_copy(zbuf, acc.at[pl.ds(s * RPT + k * 64, 64)])
                plsc.subcore_barrier()

                def body(g, carry):
                    group(g * 4, 4)
                    return carry

                lax.fori_loop(0, NCH // 4, body, 0)
                group((NCH // 4) * 4, NCH % 4)
                plsc.subcore_barrier()
                pltpu.sync_copy(acc.at[pl.ds(s * RPT, RPT)],
                                out_hbm.at[pl.ds(bh * NACC + s * RPT, RPT)])
                plsc.subcore_barrier()

    return agg_kernel(y2, srcabs_p, dst_p, zeros_hbm)


def _mm_kernel(x_ref, w_ref, s_ref, o_ref):
    o_ref[...] = jnp.dot(x_ref[...], w_ref[...],
                         preferred_element_type=jnp.float32) * s_ref[...]


def _tc_mm(x, w, sv):
    k = x.shape[1]
    return pl.pallas_call(
        _mm_kernel,
        grid=(NROW // RB,),
        in_specs=[
            pl.BlockSpec((RB, k), lambda i: (i, 0)),
            pl.BlockSpec((k, 2 * HID), lambda i: (0, 0)),
            pl.BlockSpec((RB, 1), lambda i: (i, 0)),
        ],
        out_specs=pl.BlockSpec((RB, 2 * HID), lambda i: (i, 0)),
        out_shape=jax.ShapeDtypeStruct((NROW, 2 * HID), jnp.float32),
    )(x, w, sv)


def _gate_kernel(ay_ref, s_ref, bc_ref, lz_ref, lh_ref, b2_ref, o_ref, os_ref):
    czh = ay_ref[...] * s_ref[...] + bc_ref[...]
    z = jax.nn.sigmoid(
        jnp.dot(czh[:, :HID], lz_ref[...], preferred_element_type=jnp.float32)
        + b2_ref[:, :HID])
    ht = jnp.tanh(
        jnp.dot(czh[:, HID:], lh_ref[...], preferred_element_type=jnp.float32)
        + b2_ref[:, HID:])
    out = (1.0 - z) * ht
    o_ref[...] = out
    bsum = jnp.sum(out, axis=0, keepdims=True) * 0.125
    os_ref[...] = jnp.broadcast_to(bsum, (8, HID))


def _tc_gate(ay, sv, bcat, lz, lh, b2):
    nb = NROW // RB
    return pl.pallas_call(
        _gate_kernel,
        grid=(nb,),
        in_specs=[
            pl.BlockSpec((RB, 2 * HID), lambda i: (i, 0)),
            pl.BlockSpec((RB, 1), lambda i: (i, 0)),
            pl.BlockSpec((1, 2 * HID), lambda i: (0, 0)),
            pl.BlockSpec((HID, HID), lambda i: (0, 0)),
            pl.BlockSpec((HID, HID), lambda i: (0, 0)),
            pl.BlockSpec((1, 2 * HID), lambda i: (0, 0)),
        ],
        out_specs=[
            pl.BlockSpec((RB, HID), lambda i: (i, 0)),
            pl.BlockSpec((8, HID), lambda i: (i, 0)),
        ],
        out_shape=[
            jax.ShapeDtypeStruct((NROW, HID), jnp.float32),
            jax.ShapeDtypeStruct((nb * 8, HID), jnp.float32),
        ],
    )(ay, sv, bcat, lz, lh, b2)


def _cls_kernel(c_ref, w1_ref, b1_ref, w2_ref, b2_ref, o_ref):
    h = jnp.maximum(
        jnp.dot(c_ref[...], w1_ref[...], preferred_element_type=jnp.float32)
        + b1_ref[...], 0.0)
    o_ref[...] = jnp.dot(h, w2_ref[...],
                         preferred_element_type=jnp.float32) + b2_ref[...]


def _tc_cls(comb, w1, b1, w2p, b2p):
    return pl.pallas_call(
        _cls_kernel,
        in_specs=[pl.BlockSpec(x.shape, lambda: (0,) * x.ndim)
                  for x in (comb, w1, b1, w2p, b2p)],
        out_specs=pl.BlockSpec((B, HID), lambda: (0, 0)),
        out_shape=jax.ShapeDtypeStruct((B, HID), jnp.float32),
    )(comb, w1, b1, w2p, b2p)


def kernel(x_all, mask, template_edge_index, params):
    ei = template_edge_index.astype(jnp.int32)
    loopn = jnp.arange(N, dtype=jnp.int32)
    src = jnp.concatenate([ei[0], loopn])
    dst = jnp.concatenate([ei[1], loopn])
    pad = EPAD - src.shape[0]
    src_p = jnp.concatenate([src, jnp.zeros((pad,), jnp.int32)])
    dst_p = jnp.concatenate([dst, jnp.full((pad,), N, jnp.int32)])
    src_p = src_p.reshape(16, NCH, CHUNK)
    dst_p = dst_p.reshape(16, NCH, CHUNK)
    bh = (jnp.arange(B, dtype=jnp.int32)[:, None] * (2 * N)
          + jnp.arange(2, dtype=jnp.int32)[None, :])
    srcabs = 2 * src_p[None, None] + bh[:, :, None, None, None]
    srcabs = srcabs.reshape(B * 2 * 16, NCH, CHUNK)

    ones16 = jnp.ones((CHUNK, 128), jnp.float32)
    zeros16 = jnp.zeros((64, 128), jnp.float32)
    zeros128 = jnp.zeros((64, HID), jnp.float32)

    degacc = _deg_call(dst_p, ones16, zeros16)
    deg = degacc[:N, 0]
    dinv = deg ** -0.5

    tabs = [params["emb_%d" % i] for i in range(NTAB)]
    big = jnp.concatenate(
        [jnp.pad(t, ((0, 0), (0, DPAD - t.shape[1]))) for t in tabs], axis=0)
    offs = np.concatenate([[0], np.cumsum(CARDS)[:-1]]).astype(np.int32)
    x0 = x_all[0].astype(jnp.int32)
    flat_idx = (x0 + jnp.asarray(offs)[None, :]).reshape(-1)
    flat_idx = jnp.concatenate(
        [flat_idx, jnp.zeros((IPAD - flat_idx.shape[0],), jnp.int32)])
    feats = _emb_call(big, flat_idx.reshape(32, NGCH, CHUNK))
    feats = feats[:NROW * NTAB].reshape(NROW, NTAB * DPAD)

    m0 = mask[0].astype(jnp.float32)
    sv_mask = (m0[:, None] * dinv[None, :]).reshape(NROW, 1)
    sv_plain = jnp.broadcast_to(dinv[None, :], (B, N)).reshape(NROW, 1)

    X = feats
    hs = []
    for l in range(2):
        p = params["layer_%d" % l]
        wc = jnp.concatenate([p["conv_z_W"], p["conv_h_W"]], axis=1)
        if l == 0:
            rows, o = [], 0
            for d in EMB_DIMS:
                rows.append(wc[o:o + d])
                rows.append(jnp.zeros((DPAD - d, 2 * HID), jnp.float32))
                o += d
            wc = jnp.concatenate(rows, axis=0)
            sv = sv_mask
        else:
            sv = sv_plain
        Y = _tc_mm(X, wc, sv)
        AY = _agg_call(Y.reshape(NROW * 2, HID), srcabs, dst_p, zeros128)
        AY = AY.reshape(B, 2, NACC, HID)[:, :, :N, :]
        AY = AY.transpose(0, 2, 1, 3).reshape(NROW, 2 * HID)
        bcat = jnp.concatenate([p["conv_z_b"], p["conv_h_b"]])[None, :]
        b2 = jnp.concatenate([p["lin_z_b"], p["lin_h_b"]])[None, :]
        out, osum = _tc_gate(AY, sv_plain, bcat,
                             p["lin_z_W"][:HID], p["lin_h_W"][:HID], b2)
        hs.append(osum.reshape(B, (N // RB) * 8, HID).sum(axis=1) / N)
        X = out

    comb = jnp.concatenate(hs, axis=1)
    w2p = jnp.pad(params["cls_W2"], ((0, 0), (0, HID - params["cls_W2"].shape[1])))
    b2p = jnp.pad(params["cls_b2"], (0, HID - params["cls_b2"].shape[0]))[None, :]
    res = _tc_cls(comb, params["cls_W1"], params["cls_b1"][None, :], w2p, b2p)
    return res[:, :params["cls_W2"].shape[1]]
